# Initial kernel scaffold; baseline (speedup 1.0000x reference)
import jax
import jax.numpy as jnp
from jax.experimental import pallas as pl


def kernel(x, params, edge_index):
    raise NotImplementedError("write your pallas kernel here")


# speedup vs baseline: 1.0038x; 1.0038x over previous
import jax
import jax.numpy as jnp
from jax.experimental import pallas as pl
from jax.experimental.pallas import tpu as pltpu

N_NODES = 10000
D_IN = 128
HID = 256


def _fc_body(x_ref, w_ref, b_ref, o_ref):
    o_ref[...] = jax.nn.relu(
        jnp.dot(x_ref[...], w_ref[...], preferred_element_type=jnp.float32)
        + b_ref[...]
    )


def _fc_pallas(x, W, b):
    blk = 1000
    return pl.pallas_call(
        _fc_body,
        grid=(N_NODES // blk,),
        in_specs=[
            pl.BlockSpec((blk, D_IN), lambda i: (i, 0)),
            pl.BlockSpec((D_IN, HID), lambda i: (0, 0)),
            pl.BlockSpec((1, HID), lambda i: (0, 0)),
        ],
        out_specs=pl.BlockSpec((blk, HID), lambda i: (i, 0)),
        out_shape=jax.ShapeDtypeStruct((N_NODES, HID), jnp.float32),
    )(x, W, b.reshape(1, HID))


def _layer_norm(h, g, b, eps=1e-5):
    mu = h.mean(-1, keepdims=True)
    var = h.var(-1, keepdims=True)
    return (h - mu) / jnp.sqrt(var + eps) * g + b


def _gen_conv(h, src, dst, p, n):
    msg = jax.nn.relu(h[src]) + 1e-7
    scaled = msg * p['t']
    mx = jax.ops.segment_max(scaled, dst, num_segments=n)
    mx = jnp.where(jnp.isfinite(mx), mx, 0.0)
    ex = jnp.exp(scaled - mx[dst])
    den = jax.ops.segment_sum(ex, dst, num_segments=n)
    alpha = ex / (den[dst] + 1e-16)
    agg = jax.ops.segment_sum(msg * alpha, dst, num_segments=n)
    out = agg + h
    z = out @ p['W1'] + p['b1']
    z = _layer_norm(z, p['ln_g'], p['ln_b'])
    z = jax.nn.relu(z)
    z = z @ p['W2'] + p['b2']
    return z


def kernel(x, params, edge_index):
    src = edge_index[0]
    dst = edge_index[1]
    n = x.shape[0]
    h = _fc_pallas(x, params['fc_W'], params['fc_b'])
    x_ = h
    h = _gen_conv(h, src, dst, params['conv0'], n)
    x_ = jnp.concatenate([x_, h], axis=1)
    for name in ('conv1', 'conv2'):
        p = params[name]
        hc = _gen_conv(h, src, dst, p, n)
        hc = jax.nn.relu(_layer_norm(hc, p['norm_g'], p['norm_b']))
        h = h + hc
        x_ = jnp.concatenate([x_, h], axis=1)
    h_path = jax.nn.relu(x_ @ params['phi_W'] + params['phi_b'])
    a = jnp.tanh(h_path @ params['attn_Wa'] + params['attn_ba'])
    g = jax.nn.sigmoid(h_path @ params['attn_Wb'] + params['attn_bb'])
    A = (a * g) @ params['attn_Wc'] + params['attn_bc']
    A_soft = jax.nn.softmax(A.T, axis=1)
    hp = A_soft @ h_path
    hr = jax.nn.relu(hp @ params['rho_W'] + params['rho_b']).squeeze()
    logits = (hr @ params['cls_W'] + params['cls_b'])[None, :]
    Y_hat = jax.lax.top_k(logits, 1)[1]
    return (logits, Y_hat, A_soft)


# speedup vs baseline: 2.4729x; 2.4636x over previous
import functools

import jax
import jax.numpy as jnp
from jax import lax
from jax.experimental import pallas as pl
from jax.experimental.pallas import tpu as pltpu
from jax.experimental.pallas import tpu_sc as plsc

N = 10000
E = 320000
D_IN = 128
HID = 256
CAT = 1024

NT = 16
EPB = 128
NB = 160
GRP = 8
EPAD = NT * NB * EPB
NACC = 9208
SPLIT = 8192
NBR = N - SPLIT
ZR = 576

_CLIP = 70.0


def _fc_body(x_ref, w_ref, b_ref, o_ref):
    o_ref[...] = jax.nn.relu(
        jnp.dot(x_ref[...], w_ref[...], preferred_element_type=jnp.float32)
        + b_ref[...])


def _fc(x, W, b):
    blk = 2000
    return pl.pallas_call(
        _fc_body,
        grid=(N // blk,),
        in_specs=[
            pl.BlockSpec((blk, D_IN), lambda i: (i, 0)),
            pl.BlockSpec((D_IN, HID), lambda i: (0, 0)),
            pl.BlockSpec((1, HID), lambda i: (0, 0)),
        ],
        out_specs=pl.BlockSpec((blk, HID), lambda i: (i, 0)),
        out_shape=jax.ShapeDtypeStruct((N, HID), jnp.float32),
    )(x, W, b.reshape(1, HID))


def _prep_body(h_ref, t_ref, o_ref):
    t = t_ref[0, 0]
    p = jax.nn.relu(h_ref[...]) + 1e-7
    ex = jnp.exp(jnp.minimum(p * t, _CLIP))
    q = p * ex
    o_ref[0] = ex[:, :128]
    o_ref[1] = ex[:, 128:]
    o_ref[2] = q[:, :128]
    o_ref[3] = q[:, 128:]


def _prep(h, t):
    blk = 2000
    return pl.pallas_call(
        _prep_body,
        grid=(N // blk,),
        in_specs=[
            pl.BlockSpec((blk, HID), lambda i: (i, 0)),
            pl.BlockSpec(memory_space=pltpu.MemorySpace.SMEM),
        ],
        out_specs=pl.BlockSpec((4, blk, 128), lambda i: (0, i, 0)),
        out_shape=jax.ShapeDtypeStruct((4, N, 128), jnp.float32),
    )(h, t.reshape(1, 1).astype(jnp.float32))


def _mlp_body(extra_norm, d0_ref, d1_ref, n0_ref, n1_ref, h_ref,
              w1_ref, b1_ref, lng_ref, lnb_ref, w2_ref, b2_ref,
              ng_ref, nb_ref, o_ref):
    d0, d1 = d0_ref[...], d1_ref[...]
    agg0 = jnp.where(d0 > 0, n0_ref[...] / d0, 0.0)
    agg1 = jnp.where(d1 > 0, n1_ref[...] / d1, 0.0)
    h = h_ref[...]
    out = h + jnp.concatenate([agg0, agg1], axis=1)
    z = jnp.dot(out, w1_ref[...], preferred_element_type=jnp.float32) \
        + b1_ref[...]
    mu = jnp.mean(z, axis=-1, keepdims=True)
    var = jnp.mean((z - mu) ** 2, axis=-1, keepdims=True)
    z = (z - mu) / jnp.sqrt(var + 1e-5) * lng_ref[...] + lnb_ref[...]
    z = jax.nn.relu(z)
    z = jnp.dot(z, w2_ref[...], preferred_element_type=jnp.float32) \
        + b2_ref[...]
    if extra_norm:
        mu = jnp.mean(z, axis=-1, keepdims=True)
        var = jnp.mean((z - mu) ** 2, axis=-1, keepdims=True)
        zn = (z - mu) / jnp.sqrt(var + 1e-5) * ng_ref[...] + nb_ref[...]
        z = h + jax.nn.relu(zn)
    o_ref[...] = z


def _mlp(d0, d1, n0, n1, h, p, extra_norm):
    blk = 2000
    return pl.pallas_call(
        functools.partial(_mlp_body, extra_norm),
        grid=(N // blk,),
        in_specs=[
            pl.BlockSpec((blk, 128), lambda i: (i, 0)),
            pl.BlockSpec((blk, 128), lambda i: (i, 0)),
            pl.BlockSpec((blk, 128), lambda i: (i, 0)),
            pl.BlockSpec((blk, 128), lambda i: (i, 0)),
            pl.BlockSpec((blk, HID), lambda i: (i, 0)),
            pl.BlockSpec((HID, 2 * HID), lambda i: (0, 0)),
            pl.BlockSpec((1, 2 * HID), lambda i: (0, 0)),
            pl.BlockSpec((1, 2 * HID), lambda i: (0, 0)),
            pl.BlockSpec((1, 2 * HID), lambda i: (0, 0)),
            pl.BlockSpec((2 * HID, HID), lambda i: (0, 0)),
            pl.BlockSpec((1, HID), lambda i: (0, 0)),
            pl.BlockSpec((1, HID), lambda i: (0, 0)),
            pl.BlockSpec((1, HID), lambda i: (0, 0)),
        ],
        out_specs=pl.BlockSpec((blk, HID), lambda i: (i, 0)),
        out_shape=jax.ShapeDtypeStruct((N, HID), jnp.float32),
    )(d0, d1, n0, n1, h,
      p['W1'], p['b1'].reshape(1, -1),
      p['ln_g'].reshape(1, -1), p['ln_b'].reshape(1, -1),
      p['W2'], p['b2'].reshape(1, -1),
      p.get('norm_g', p['b2']).reshape(1, -1),
      p.get('norm_b', p['b2']).reshape(1, -1))


def _attn_body(x_ref, pw_ref, pb_ref, wa_ref, ba_ref, wb_ref, bb_ref,
               wc_ref, bc_ref, ex_ref, num_ref, den_ref):
    i = pl.program_id(0)

    @pl.when(i == 0)
    def _():
        num_ref[...] = jnp.zeros_like(num_ref)
        den_ref[...] = jnp.zeros_like(den_ref)

    hp = jax.nn.relu(
        jnp.dot(x_ref[...], pw_ref[...], preferred_element_type=jnp.float32)
        + pb_ref[...])
    za = jnp.dot(hp, wa_ref[...], preferred_element_type=jnp.float32) \
        + ba_ref[...]
    a = 1.0 - 2.0 / (jnp.exp(2.0 * za) + 1.0)
    zb = jnp.dot(hp, wb_ref[...], preferred_element_type=jnp.float32) \
        + bb_ref[...]
    g = 1.0 / (1.0 + jnp.exp(-zb))
    A = jnp.dot(a * g, wc_ref[...], preferred_element_type=jnp.float32) \
        + bc_ref[...]
    ex = jnp.exp(A[:, 0:1])
    ex_ref[...] = ex
    num_ref[...] += jnp.sum(hp * ex, axis=0, keepdims=True)
    den_ref[...] += jnp.broadcast_to(jnp.sum(ex), (1, 128))


def _attn(x_, pW, pb, Wa, ba, Wb, bb, Wc_pad, bc_pad):
    blk = 1000
    return pl.pallas_call(
        _attn_body,
        grid=(N // blk,),
        in_specs=[
            pl.BlockSpec((blk, CAT), lambda i: (i, 0)),
            pl.BlockSpec((CAT, CAT), lambda i: (0, 0)),
            pl.BlockSpec((1, CAT), lambda i: (0, 0)),
            pl.BlockSpec((CAT, CAT), lambda i: (0, 0)),
            pl.BlockSpec((1, CAT), lambda i: (0, 0)),
            pl.BlockSpec((CAT, CAT), lambda i: (0, 0)),
            pl.BlockSpec((1, CAT), lambda i: (0, 0)),
            pl.BlockSpec((CAT, 128), lambda i: (0, 0)),
            pl.BlockSpec((1, 128), lambda i: (0, 0)),
        ],
        out_specs=[
            pl.BlockSpec((blk, 1), lambda i: (i, 0)),
            pl.BlockSpec((1, CAT), lambda i: (0, 0)),
            pl.BlockSpec((1, 128), lambda i: (0, 0)),
        ],
        out_shape=[
            jax.ShapeDtypeStruct((N, 1), jnp.float32),
            jax.ShapeDtypeStruct((1, CAT), jnp.float32),
            jax.ShapeDtypeStruct((1, 128), jnp.float32),
        ],
    )(x_, pW, pb.reshape(1, -1), Wa, ba.reshape(1, -1),
      Wb, bb.reshape(1, -1), Wc_pad, bc_pad)


def _head_body(num_ref, den_ref, ex_ref, rw_ref, rb_ref, cw_ref, cb_ref,
               lg_ref, yh_ref, as_ref):
    den = den_ref[...][0:1, 0:1]
    hp = num_ref[...] / den
    hr = jax.nn.relu(
        jnp.dot(hp, rw_ref[...], preferred_element_type=jnp.float32)
        + rb_ref[...])
    lg = jnp.dot(hr, cw_ref[...], preferred_element_type=jnp.float32) \
        + cb_ref[...]
    lg_ref[...] = lg
    lane = lax.broadcasted_iota(jnp.int32, (1, 128), 1)
    l0 = jnp.sum(jnp.where(lane == 0, lg, 0.0))
    l1 = jnp.sum(jnp.where(lane == 1, lg, 0.0))
    yh_ref[...] = jnp.where(l1 > l0, 1, 0).astype(jnp.int32).reshape(1, 1)
    as_ref[...] = ex_ref[...] / den


def _head(num, den, exA, rW, rb, cW_pad, cb_pad):
    return pl.pallas_call(
        _head_body,
        out_shape=[
            jax.ShapeDtypeStruct((1, 128), jnp.float32),
            jax.ShapeDtypeStruct((1, 1), jnp.int32),
            jax.ShapeDtypeStruct((N, 1), jnp.float32),
        ],
    )(num, den, exA, rW, rb.reshape(1, -1), cW_pad, cb_pad)


_MESH = plsc.VectorSubcoreMesh(core_axis_name="c", subcore_axis_name="s")


def _seg_body(tabs, src_hbm, dstab_hbm, outs, src8, ids8, buf, acc):
    c = lax.axis_index("c")
    s = lax.axis_index("s")

    def zero_acc():
        @pl.loop(0, EPB)
        def _(r):
            for j in range(128 // 16):
                buf[r, pl.ds(j * 16, 16)] = jnp.zeros((16,), jnp.float32)

        @pl.when(s < NT - 1)
        def _():
            @pl.loop(0, ZR // 128)
            def _(r):
                pltpu.sync_copy(buf, acc.at[pl.ds(s * ZR + r * 128, 128)])

            pltpu.sync_copy(buf.at[pl.ds(0, ZR % 128)],
                            acc.at[pl.ds(s * ZR + (ZR // 128) * 128,
                                         ZR % 128)])

        @pl.when(s == NT - 1)
        def _():
            last = NACC - (NT - 1) * ZR

            @pl.loop(0, last // 128)
            def _(r):
                pltpu.sync_copy(
                    buf, acc.at[pl.ds((NT - 1) * ZR + r * 128, 128)])

            pltpu.sync_copy(
                buf.at[pl.ds(0, last % 128)],
                acc.at[pl.ds((NT - 1) * ZR + (last // 128) * 128,
                             last % 128)])

    def one_pass(k, pass_b, flush):
        tab = tabs.at[k]
        zero_acc()
        plsc.subcore_barrier()

        @pl.loop(0, NB // GRP)
        def _(g):
            pltpu.sync_copy(src_hbm.at[s].at[pl.ds(g * GRP, GRP)], src8)
            pltpu.sync_copy(dstab_hbm.at[s].at[pl.ds(g * GRP, GRP)], ids8)
            for r in range(GRP):
                for j in range(EPB // 16):
                    v = ids8[r, pl.ds(j * 16, 16)]
                    if pass_b:
                        ids8[r, pl.ds(j * 16, 16)] = v >> 16
                    else:
                        ids8[r, pl.ds(j * 16, 16)] = v & 0xFFFF
            for r in range(GRP):
                pltpu.sync_copy(tab.at[src8.at[r]], buf)
                pltpu.sync_copy(buf, acc.at[ids8.at[r]], add=True)

        plsc.subcore_barrier()
        flush(k)
        plsc.subcore_barrier()

    def flush_a(k):
        pltpu.sync_copy(acc.at[pl.ds(s * 512, 512)],
                        outs.at[k].at[pl.ds(s * 512, 512)])

    def flush_b(k):
        @pl.when(s < 14)
        def _():
            pltpu.sync_copy(acc.at[pl.ds(s * 128, 128)],
                            outs.at[k].at[pl.ds(SPLIT + s * 128, 128)])

        @pl.when(s == 14)
        def _():
            pltpu.sync_copy(acc.at[pl.ds(1792, 16)],
                            outs.at[k].at[pl.ds(SPLIT + 1792, 16)])

    @pl.when(c == 0)
    def _():
        for k in range(2):
            one_pass(k, False, flush_a)
            one_pass(k, True, flush_b)

    @pl.when(c == 1)
    def _():
        for k in range(2, 4):
            one_pass(k, False, flush_a)
            one_pass(k, True, flush_b)


@jax.jit
def _seg_sums(tabs, src_t, dstab_t):
    f = pl.kernel(
        _seg_body,
        out_type=jax.ShapeDtypeStruct((4, N, 128), jnp.float32),
        mesh=_MESH,
        scratch_types=[
            pltpu.VMEM((GRP, EPB), jnp.int32),
            pltpu.VMEM((GRP, EPB), jnp.int32),
            pltpu.VMEM((EPB, 128), jnp.float32),
            pltpu.VMEM_SHARED((NACC, 128), jnp.float32),
        ],
    )
    return f(tabs, src_t, dstab_t)


def kernel(x, params, edge_index):
    src = edge_index[0]
    dst = edge_index[1]
    pad = EPAD - E
    src_p = jnp.concatenate([src, jnp.zeros((pad,), jnp.int32)])
    dst_p = jnp.concatenate(
        [dst, jnp.full((pad,), N, jnp.int32)])
    dsta_p = jnp.where(dst_p < SPLIT, dst_p, SPLIT + dst_p % (NACC - SPLIT))
    dstb_p = jnp.where((dst_p >= SPLIT) & (dst_p < N), dst_p - SPLIT,
                       NBR + dst_p % (NACC - NBR))
    dstab_p = dsta_p | (dstb_p << 16)
    src_t = src_p.reshape(NT, NB, EPB)
    dstab_t = dstab_p.reshape(NT, NB, EPB)

    h = _fc(x, params['fc_W'], params['fc_b'])
    feats = [h]
    for li, name in enumerate(('conv0', 'conv1', 'conv2')):
        p = params[name]
        tabs = _prep(h, p['t'])
        acc = _seg_sums(tabs, src_t, dstab_t)
        h = _mlp(acc[0], acc[1], acc[2], acc[3], h, p,
                 extra_norm=(li > 0))
        feats.append(h)

    x_ = jnp.concatenate(feats, axis=1)
    Wc_pad = jnp.pad(params['attn_Wc'], ((0, 0), (0, 127)))
    bc_pad = jnp.pad(params['attn_bc'], (0, 127)).reshape(1, 128)
    exA, num, den = _attn(x_, params['phi_W'], params['phi_b'],
                          params['attn_Wa'], params['attn_ba'],
                          params['attn_Wb'], params['attn_bb'],
                          Wc_pad, bc_pad)
    cW_pad = jnp.pad(params['cls_W'], ((0, 0), (0, 126)))
    cb_pad = jnp.pad(params['cls_b'], (0, 126)).reshape(1, 128)
    lg_pad, yhat, asoft_col = _head(num, den, exA, params['rho_W'],
                                    params['rho_b'], cW_pad, cb_pad)
    logits = lg_pad[:, :2]
    A_soft = asoft_col.T
    return (logits, yhat, A_soft)


# speedup vs baseline: 2.8931x; 1.1699x over previous
import functools

import jax
import jax.numpy as jnp
from jax import lax
from jax.experimental import pallas as pl
from jax.experimental.pallas import tpu as pltpu
from jax.experimental.pallas import tpu_sc as plsc

N = 10000
E = 320000
D_IN = 128
HID = 256
CAT = 1024

NT = 16
EPB = 128
NB = 160
GRP = 16
NGRP = NB // GRP
EPAD = NT * NB * EPB
NACC = 9208
SPLIT = 8192
NBR = N - SPLIT
ZR = 576

_CLIP = 70.0


def _fc_body(x_ref, w_ref, b_ref, o_ref):
    o_ref[...] = jax.nn.relu(
        jnp.dot(x_ref[...], w_ref[...], preferred_element_type=jnp.float32)
        + b_ref[...])


def _fc(x, W, b):
    blk = 2000
    return pl.pallas_call(
        _fc_body,
        grid=(N // blk,),
        in_specs=[
            pl.BlockSpec((blk, D_IN), lambda i: (i, 0)),
            pl.BlockSpec((D_IN, HID), lambda i: (0, 0)),
            pl.BlockSpec((1, HID), lambda i: (0, 0)),
        ],
        out_specs=pl.BlockSpec((blk, HID), lambda i: (i, 0)),
        out_shape=jax.ShapeDtypeStruct((N, HID), jnp.float32),
    )(x, W, b.reshape(1, HID))


def _prep_body(h_ref, t_ref, o_ref):
    t = t_ref[0, 0]
    p = jax.nn.relu(h_ref[...]) + 1e-7
    ex = jnp.exp(jnp.minimum(p * t, _CLIP))
    q = p * ex
    o_ref[0] = ex[:, :128]
    o_ref[1] = ex[:, 128:]
    o_ref[2] = q[:, :128]
    o_ref[3] = q[:, 128:]


def _prep(h, t):
    blk = 2000
    return pl.pallas_call(
        _prep_body,
        grid=(N // blk,),
        in_specs=[
            pl.BlockSpec((blk, HID), lambda i: (i, 0)),
            pl.BlockSpec(memory_space=pltpu.MemorySpace.SMEM),
        ],
        out_specs=pl.BlockSpec((4, blk, 128), lambda i: (0, i, 0)),
        out_shape=jax.ShapeDtypeStruct((4, N, 128), jnp.float32),
    )(h, t.reshape(1, 1).astype(jnp.float32))


def _mlp_body(extra_norm, d0_ref, d1_ref, n0_ref, n1_ref, h_ref,
              w1_ref, b1_ref, lng_ref, lnb_ref, w2_ref, b2_ref,
              ng_ref, nb_ref, o_ref):
    d0, d1 = d0_ref[...], d1_ref[...]
    agg0 = jnp.where(d0 > 0, n0_ref[...] / d0, 0.0)
    agg1 = jnp.where(d1 > 0, n1_ref[...] / d1, 0.0)
    h = h_ref[...]
    out = h + jnp.concatenate([agg0, agg1], axis=1)
    z = jnp.dot(out, w1_ref[...], preferred_element_type=jnp.float32) \
        + b1_ref[...]
    mu = jnp.mean(z, axis=-1, keepdims=True)
    var = jnp.mean((z - mu) ** 2, axis=-1, keepdims=True)
    z = (z - mu) / jnp.sqrt(var + 1e-5) * lng_ref[...] + lnb_ref[...]
    z = jax.nn.relu(z)
    z = jnp.dot(z, w2_ref[...], preferred_element_type=jnp.float32) \
        + b2_ref[...]
    if extra_norm:
        mu = jnp.mean(z, axis=-1, keepdims=True)
        var = jnp.mean((z - mu) ** 2, axis=-1, keepdims=True)
        zn = (z - mu) / jnp.sqrt(var + 1e-5) * ng_ref[...] + nb_ref[...]
        z = h + jax.nn.relu(zn)
    o_ref[...] = z


def _mlp(d0, d1, n0, n1, h, p, extra_norm):
    blk = 2000
    return pl.pallas_call(
        functools.partial(_mlp_body, extra_norm),
        grid=(N // blk,),
        in_specs=[
            pl.BlockSpec((blk, 128), lambda i: (i, 0)),
            pl.BlockSpec((blk, 128), lambda i: (i, 0)),
            pl.BlockSpec((blk, 128), lambda i: (i, 0)),
            pl.BlockSpec((blk, 128), lambda i: (i, 0)),
            pl.BlockSpec((blk, HID), lambda i: (i, 0)),
            pl.BlockSpec((HID, 2 * HID), lambda i: (0, 0)),
            pl.BlockSpec((1, 2 * HID), lambda i: (0, 0)),
            pl.BlockSpec((1, 2 * HID), lambda i: (0, 0)),
            pl.BlockSpec((1, 2 * HID), lambda i: (0, 0)),
            pl.BlockSpec((2 * HID, HID), lambda i: (0, 0)),
            pl.BlockSpec((1, HID), lambda i: (0, 0)),
            pl.BlockSpec((1, HID), lambda i: (0, 0)),
            pl.BlockSpec((1, HID), lambda i: (0, 0)),
        ],
        out_specs=pl.BlockSpec((blk, HID), lambda i: (i, 0)),
        out_shape=jax.ShapeDtypeStruct((N, HID), jnp.float32),
    )(d0, d1, n0, n1, h,
      p['W1'], p['b1'].reshape(1, -1),
      p['ln_g'].reshape(1, -1), p['ln_b'].reshape(1, -1),
      p['W2'], p['b2'].reshape(1, -1),
      p.get('norm_g', p['b2']).reshape(1, -1),
      p.get('norm_b', p['b2']).reshape(1, -1))


def _attn_body(x_ref, pw_ref, pb_ref, wa_ref, ba_ref, wb_ref, bb_ref,
               wc_ref, bc_ref, ex_ref, num_ref, den_ref):
    i = pl.program_id(0)

    @pl.when(i == 0)
    def _():
        num_ref[...] = jnp.zeros_like(num_ref)
        den_ref[...] = jnp.zeros_like(den_ref)

    hp = jax.nn.relu(
        jnp.dot(x_ref[...], pw_ref[...], preferred_element_type=jnp.float32)
        + pb_ref[...])
    za = jnp.dot(hp, wa_ref[...], preferred_element_type=jnp.float32) \
        + ba_ref[...]
    a = 1.0 - 2.0 / (jnp.exp(2.0 * za) + 1.0)
    zb = jnp.dot(hp, wb_ref[...], preferred_element_type=jnp.float32) \
        + bb_ref[...]
    g = 1.0 / (1.0 + jnp.exp(-zb))
    A = jnp.dot(a * g, wc_ref[...], preferred_element_type=jnp.float32) \
        + bc_ref[...]
    ex = jnp.exp(A[:, 0:1])
    ex_ref[...] = ex
    num_ref[...] += jnp.sum(hp * ex, axis=0, keepdims=True)
    den_ref[...] += jnp.broadcast_to(jnp.sum(ex), (1, 128))


def _attn(x_, pW, pb, Wa, ba, Wb, bb, Wc_pad, bc_pad):
    blk = 1000
    return pl.pallas_call(
        _attn_body,
        grid=(N // blk,),
        in_specs=[
            pl.BlockSpec((blk, CAT), lambda i: (i, 0)),
            pl.BlockSpec((CAT, CAT), lambda i: (0, 0)),
            pl.BlockSpec((1, CAT), lambda i: (0, 0)),
            pl.BlockSpec((CAT, CAT), lambda i: (0, 0)),
            pl.BlockSpec((1, CAT), lambda i: (0, 0)),
            pl.BlockSpec((CAT, CAT), lambda i: (0, 0)),
            pl.BlockSpec((1, CAT), lambda i: (0, 0)),
            pl.BlockSpec((CAT, 128), lambda i: (0, 0)),
            pl.BlockSpec((1, 128), lambda i: (0, 0)),
        ],
        out_specs=[
            pl.BlockSpec((blk, 1), lambda i: (i, 0)),
            pl.BlockSpec((1, CAT), lambda i: (0, 0)),
            pl.BlockSpec((1, 128), lambda i: (0, 0)),
        ],
        out_shape=[
            jax.ShapeDtypeStruct((N, 1), jnp.float32),
            jax.ShapeDtypeStruct((1, CAT), jnp.float32),
            jax.ShapeDtypeStruct((1, 128), jnp.float32),
        ],
    )(x_, pW, pb.reshape(1, -1), Wa, ba.reshape(1, -1),
      Wb, bb.reshape(1, -1), Wc_pad, bc_pad)


def _head_body(num_ref, den_ref, ex_ref, rw_ref, rb_ref, cw_ref, cb_ref,
               lg_ref, yh_ref, as_ref):
    den = den_ref[...][0:1, 0:1]
    hp = num_ref[...] / den
    hr = jax.nn.relu(
        jnp.dot(hp, rw_ref[...], preferred_element_type=jnp.float32)
        + rb_ref[...])
    lg = jnp.dot(hr, cw_ref[...], preferred_element_type=jnp.float32) \
        + cb_ref[...]
    lg_ref[...] = lg
    lane = lax.broadcasted_iota(jnp.int32, (1, 128), 1)
    l0 = jnp.sum(jnp.where(lane == 0, lg, 0.0))
    l1 = jnp.sum(jnp.where(lane == 1, lg, 0.0))
    yh_ref[...] = jnp.where(l1 > l0, 1, 0).astype(jnp.int32).reshape(1, 1)
    as_ref[...] = ex_ref[...] / den


def _head(num, den, exA, rW, rb, cW_pad, cb_pad):
    return pl.pallas_call(
        _head_body,
        out_shape=[
            jax.ShapeDtypeStruct((1, 128), jnp.float32),
            jax.ShapeDtypeStruct((1, 1), jnp.int32),
            jax.ShapeDtypeStruct((N, 1), jnp.float32),
        ],
    )(num, den, exA, rW, rb.reshape(1, -1), cW_pad, cb_pad)


_MESH = plsc.VectorSubcoreMesh(core_axis_name="c", subcore_axis_name="s")


def _seg_body(tabs, src_hbm, dstab_hbm, outs,
              srcb0, srcb1, idsb0, idsb1, g0, g1,
              sg0, sg1, ss0, ss1, si0, si1, acc):
    c = lax.axis_index("c")
    s = lax.axis_index("s")
    srcb = [srcb0, srcb1]
    idsb = [idsb0, idsb1]
    G = [g0, g1]
    sg = [sg0, sg1]
    ss = [ss0, ss1]
    si = [si0, si1]
    src_hb = src_hbm.at[s]
    ab_hb = dstab_hbm.at[s]

    def zero_acc():
        @pl.loop(0, EPB)
        def _(r):
            for j in range(128 // 16):
                g0[r, pl.ds(j * 16, 16)] = jnp.zeros((16,), jnp.float32)

        @pl.when(s < NT - 1)
        def _():
            @pl.loop(0, ZR // 128)
            def _(r):
                pltpu.sync_copy(g0, acc.at[pl.ds(s * ZR + r * 128, 128)])

            pltpu.sync_copy(g0.at[pl.ds(0, ZR % 128)],
                            acc.at[pl.ds(s * ZR + (ZR // 128) * 128,
                                         ZR % 128)])

        @pl.when(s == NT - 1)
        def _():
            last = NACC - (NT - 1) * ZR

            @pl.loop(0, last // 128)
            def _(r):
                pltpu.sync_copy(
                    g0, acc.at[pl.ds((NT - 1) * ZR + r * 128, 128)])

            pltpu.sync_copy(
                g0.at[pl.ds(0, last % 128)],
                acc.at[pl.ds((NT - 1) * ZR + (last // 128) * 128,
                             last % 128)])

    def idx_prefetch(g, par):
        pltpu.async_copy(src_hb.at[pl.ds(g * GRP, GRP)], srcb[par], si[par])
        pltpu.async_copy(ab_hb.at[pl.ds(g * GRP, GRP)], idsb[par], si[par])

    def one_pass(k, pass_b, flush):
        tab = tabs.at[k]
        zero_acc()
        plsc.subcore_barrier()
        idx_prefetch(0, 0)

        def do_group(g, par):
            @pl.when(g + 1 < NGRP)
            def _():
                idx_prefetch(g + 1, 1 - par)

            src8, ids8 = srcb[par], idsb[par]
            pltpu.make_async_copy(src_hb.at[pl.ds(0, GRP)], src8,
                                  si[par]).wait()
            pltpu.make_async_copy(ab_hb.at[pl.ds(0, GRP)], ids8,
                                  si[par]).wait()
            for r in range(GRP):
                for j in range(EPB // 16):
                    v = ids8[r, pl.ds(j * 16, 16)]
                    if pass_b:
                        ids8[r, pl.ds(j * 16, 16)] = v >> 16
                    else:
                        ids8[r, pl.ds(j * 16, 16)] = v & 0xFFFF
            pltpu.async_copy(tab.at[src8.at[0]], G[0], sg[0])
            for r in range(GRP):
                q = r % 2
                pltpu.make_async_copy(tab.at[src8.at[r]], G[q],
                                      sg[q]).wait()
                if r + 1 < GRP:
                    if r >= 1:
                        pltpu.make_async_copy(
                            G[1 - q], acc.at[ids8.at[r - 1]],
                            ss[1 - q]).wait()
                    pltpu.async_copy(tab.at[src8.at[r + 1]], G[1 - q],
                                     sg[1 - q])
                pltpu.async_copy(G[q], acc.at[ids8.at[r]], ss[q],
                                 add=True)
            pltpu.make_async_copy(G[0], acc.at[ids8.at[GRP - 2]],
                                  ss[0]).wait()
            pltpu.make_async_copy(G[1], acc.at[ids8.at[GRP - 1]],
                                  ss[1]).wait()

        @pl.loop(0, NGRP // 2)
        def _(gp):
            do_group(2 * gp, 0)
            do_group(2 * gp + 1, 1)

        plsc.subcore_barrier()
        flush(k)
        plsc.subcore_barrier()

    def flush_a(k):
        pltpu.sync_copy(acc.at[pl.ds(s * 512, 512)],
                        outs.at[k].at[pl.ds(s * 512, 512)])

    def flush_b(k):
        @pl.when(s < 14)
        def _():
            pltpu.sync_copy(acc.at[pl.ds(s * 128, 128)],
                            outs.at[k].at[pl.ds(SPLIT + s * 128, 128)])

        @pl.when(s == 14)
        def _():
            pltpu.sync_copy(acc.at[pl.ds(1792, 16)],
                            outs.at[k].at[pl.ds(SPLIT + 1792, 16)])

    @pl.when(c == 0)
    def _():
        for k in range(2):
            one_pass(k, False, flush_a)
            one_pass(k, True, flush_b)

    @pl.when(c == 1)
    def _():
        for k in range(2, 4):
            one_pass(k, False, flush_a)
            one_pass(k, True, flush_b)


def _seg_sums(tabs, src_t, dstab_t):
    f = pl.kernel(
        _seg_body,
        out_type=jax.ShapeDtypeStruct((4, N, 128), jnp.float32),
        mesh=_MESH,
        scratch_types=[
            pltpu.VMEM((GRP, EPB), jnp.int32),
            pltpu.VMEM((GRP, EPB), jnp.int32),
            pltpu.VMEM((GRP, EPB), jnp.int32),
            pltpu.VMEM((GRP, EPB), jnp.int32),
            pltpu.VMEM((EPB, 128), jnp.float32),
            pltpu.VMEM((EPB, 128), jnp.float32),
            pltpu.SemaphoreType.DMA,
            pltpu.SemaphoreType.DMA,
            pltpu.SemaphoreType.DMA,
            pltpu.SemaphoreType.DMA,
            pltpu.SemaphoreType.DMA,
            pltpu.SemaphoreType.DMA,
            pltpu.VMEM_SHARED((NACC, 128), jnp.float32),
        ],
    )
    return f(tabs, src_t, dstab_t)


def kernel(x, params, edge_index):
    src = edge_index[0]
    dst = edge_index[1]
    pad = EPAD - E
    src_p = jnp.concatenate([src, jnp.zeros((pad,), jnp.int32)])
    dst_p = jnp.concatenate(
        [dst, jnp.full((pad,), N, jnp.int32)])
    dsta_p = jnp.where(dst_p < SPLIT, dst_p, SPLIT + dst_p % (NACC - SPLIT))
    dstb_p = jnp.where((dst_p >= SPLIT) & (dst_p < N), dst_p - SPLIT,
                       NBR + dst_p % (NACC - NBR))
    dstab_p = dsta_p | (dstb_p << 16)
    src_t = src_p.reshape(NT, NB, EPB)
    dstab_t = dstab_p.reshape(NT, NB, EPB)

    h = _fc(x, params['fc_W'], params['fc_b'])
    feats = [h]
    for li, name in enumerate(('conv0', 'conv1', 'conv2')):
        p = params[name]
        tabs = _prep(h, p['t'])
        acc = _seg_sums(tabs, src_t, dstab_t)
        h = _mlp(acc[0], acc[1], acc[2], acc[3], h, p,
                 extra_norm=(li > 0))
        feats.append(h)

    x_ = jnp.concatenate(feats, axis=1)
    Wc_pad = jnp.pad(params['attn_Wc'], ((0, 0), (0, 127)))
    bc_pad = jnp.pad(params['attn_bc'], (0, 127)).reshape(1, 128)
    exA, num, den = _attn(x_, params['phi_W'], params['phi_b'],
                          params['attn_Wa'], params['attn_ba'],
                          params['attn_Wb'], params['attn_bb'],
                          Wc_pad, bc_pad)
    cW_pad = jnp.pad(params['cls_W'], ((0, 0), (0, 126)))
    cb_pad = jnp.pad(params['cls_b'], (0, 126)).reshape(1, 128)
    lg_pad, yhat, asoft_col = _head(num, den, exA, params['rho_W'],
                                    params['rho_b'], cW_pad, cb_pad)
    logits = lg_pad[:, :2]
    A_soft = asoft_col.T
    return (logits, yhat, A_soft)


# speedup vs baseline: 5.5315x; 1.9120x over previous
import functools

import jax
import jax.numpy as jnp
from jax import lax
from jax.experimental import pallas as pl
from jax.experimental.pallas import tpu as pltpu
from jax.experimental.pallas import tpu_sc as plsc

N = 10000
E = 320000
D_IN = 128
HID = 256
CAT = 1024

NT = 16
EPB = 128
NB = 160
GRP = 16
NGRP = NB // GRP
EPAD = NT * NB * EPB
NACC = 10112
ZR = 632

_CLIP = 70.0


def _fc_body(x_ref, w_ref, b_ref, o_ref):
    o_ref[...] = jax.nn.relu(
        jnp.dot(x_ref[...], w_ref[...], preferred_element_type=jnp.float32)
        + b_ref[...])


def _fc(x, W, b):
    blk = 2000
    return pl.pallas_call(
        _fc_body,
        grid=(N // blk,),
        in_specs=[
            pl.BlockSpec((blk, D_IN), lambda i: (i, 0)),
            pl.BlockSpec((D_IN, HID), lambda i: (0, 0)),
            pl.BlockSpec((1, HID), lambda i: (0, 0)),
        ],
        out_specs=pl.BlockSpec((blk, HID), lambda i: (i, 0)),
        out_shape=jax.ShapeDtypeStruct((N, HID), jnp.float32),
    )(x, W, b.reshape(1, HID))


def _prep_body(h_ref, t_ref, o_ref):
    t = t_ref[0, 0]
    p = jax.nn.relu(h_ref[...]) + 1e-7
    ex = jnp.exp(jnp.minimum(p * t, _CLIP))
    q = p * ex
    o_ref[0] = ex[:, :128]
    o_ref[1] = ex[:, 128:]
    o_ref[2] = q[:, :128]
    o_ref[3] = q[:, 128:]


def _prep(h, t):
    blk = 2000
    return pl.pallas_call(
        _prep_body,
        grid=(N // blk,),
        in_specs=[
            pl.BlockSpec((blk, HID), lambda i: (i, 0)),
            pl.BlockSpec(memory_space=pltpu.MemorySpace.SMEM),
        ],
        out_specs=pl.BlockSpec((4, blk, 128), lambda i: (0, i, 0)),
        out_shape=jax.ShapeDtypeStruct((4, N, 128), jnp.float32),
    )(h, t.reshape(1, 1).astype(jnp.float32))


def _mlp_body(extra_norm, d0_ref, d1_ref, n0_ref, n1_ref, h_ref,
              w1_ref, b1_ref, lng_ref, lnb_ref, w2_ref, b2_ref,
              ng_ref, nb_ref, o_ref):
    d0, d1 = d0_ref[...], d1_ref[...]
    agg0 = jnp.where(d0 > 0, n0_ref[...] / d0, 0.0)
    agg1 = jnp.where(d1 > 0, n1_ref[...] / d1, 0.0)
    h = h_ref[...]
    out = h + jnp.concatenate([agg0, agg1], axis=1)
    z = jnp.dot(out, w1_ref[...], preferred_element_type=jnp.float32) \
        + b1_ref[...]
    mu = jnp.mean(z, axis=-1, keepdims=True)
    var = jnp.mean((z - mu) ** 2, axis=-1, keepdims=True)
    z = (z - mu) / jnp.sqrt(var + 1e-5) * lng_ref[...] + lnb_ref[...]
    z = jax.nn.relu(z)
    z = jnp.dot(z, w2_ref[...], preferred_element_type=jnp.float32) \
        + b2_ref[...]
    if extra_norm:
        mu = jnp.mean(z, axis=-1, keepdims=True)
        var = jnp.mean((z - mu) ** 2, axis=-1, keepdims=True)
        zn = (z - mu) / jnp.sqrt(var + 1e-5) * ng_ref[...] + nb_ref[...]
        z = h + jax.nn.relu(zn)
    o_ref[...] = z


def _mlp(d0, d1, n0, n1, h, p, extra_norm):
    blk = 2000
    return pl.pallas_call(
        functools.partial(_mlp_body, extra_norm),
        grid=(N // blk,),
        in_specs=[
            pl.BlockSpec((blk, 128), lambda i: (i, 0)),
            pl.BlockSpec((blk, 128), lambda i: (i, 0)),
            pl.BlockSpec((blk, 128), lambda i: (i, 0)),
            pl.BlockSpec((blk, 128), lambda i: (i, 0)),
            pl.BlockSpec((blk, HID), lambda i: (i, 0)),
            pl.BlockSpec((HID, 2 * HID), lambda i: (0, 0)),
            pl.BlockSpec((1, 2 * HID), lambda i: (0, 0)),
            pl.BlockSpec((1, 2 * HID), lambda i: (0, 0)),
            pl.BlockSpec((1, 2 * HID), lambda i: (0, 0)),
            pl.BlockSpec((2 * HID, HID), lambda i: (0, 0)),
            pl.BlockSpec((1, HID), lambda i: (0, 0)),
            pl.BlockSpec((1, HID), lambda i: (0, 0)),
            pl.BlockSpec((1, HID), lambda i: (0, 0)),
        ],
        out_specs=pl.BlockSpec((blk, HID), lambda i: (i, 0)),
        out_shape=jax.ShapeDtypeStruct((N, HID), jnp.float32),
    )(d0, d1, n0, n1, h,
      p['W1'], p['b1'].reshape(1, -1),
      p['ln_g'].reshape(1, -1), p['ln_b'].reshape(1, -1),
      p['W2'], p['b2'].reshape(1, -1),
      p.get('norm_g', p['b2']).reshape(1, -1),
      p.get('norm_b', p['b2']).reshape(1, -1))


def _attn_body(x_ref, pw_ref, pb_ref, wa_ref, ba_ref, wb_ref, bb_ref,
               wc_ref, bc_ref, ex_ref, num_ref, den_ref):
    i = pl.program_id(0)

    @pl.when(i == 0)
    def _():
        num_ref[...] = jnp.zeros_like(num_ref)
        den_ref[...] = jnp.zeros_like(den_ref)

    hp = jax.nn.relu(
        jnp.dot(x_ref[...], pw_ref[...], preferred_element_type=jnp.float32)
        + pb_ref[...])
    za = jnp.dot(hp, wa_ref[...], preferred_element_type=jnp.float32) \
        + ba_ref[...]
    a = 1.0 - 2.0 / (jnp.exp(2.0 * za) + 1.0)
    zb = jnp.dot(hp, wb_ref[...], preferred_element_type=jnp.float32) \
        + bb_ref[...]
    g = 1.0 / (1.0 + jnp.exp(-zb))
    A = jnp.dot(a * g, wc_ref[...], preferred_element_type=jnp.float32) \
        + bc_ref[...]
    ex = jnp.exp(A[:, 0:1])
    ex_ref[...] = ex
    num_ref[...] += jnp.sum(hp * ex, axis=0, keepdims=True)
    den_ref[...] += jnp.broadcast_to(jnp.sum(ex), (1, 128))


def _attn(x_, pW, pb, Wa, ba, Wb, bb, Wc_pad, bc_pad):
    blk = 1000
    return pl.pallas_call(
        _attn_body,
        grid=(N // blk,),
        in_specs=[
            pl.BlockSpec((blk, CAT), lambda i: (i, 0)),
            pl.BlockSpec((CAT, CAT), lambda i: (0, 0)),
            pl.BlockSpec((1, CAT), lambda i: (0, 0)),
            pl.BlockSpec((CAT, CAT), lambda i: (0, 0)),
            pl.BlockSpec((1, CAT), lambda i: (0, 0)),
            pl.BlockSpec((CAT, CAT), lambda i: (0, 0)),
            pl.BlockSpec((1, CAT), lambda i: (0, 0)),
            pl.BlockSpec((CAT, 128), lambda i: (0, 0)),
            pl.BlockSpec((1, 128), lambda i: (0, 0)),
        ],
        out_specs=[
            pl.BlockSpec((blk, 1), lambda i: (i, 0)),
            pl.BlockSpec((1, CAT), lambda i: (0, 0)),
            pl.BlockSpec((1, 128), lambda i: (0, 0)),
        ],
        out_shape=[
            jax.ShapeDtypeStruct((N, 1), jnp.float32),
            jax.ShapeDtypeStruct((1, CAT), jnp.float32),
            jax.ShapeDtypeStruct((1, 128), jnp.float32),
        ],
    )(x_, pW, pb.reshape(1, -1), Wa, ba.reshape(1, -1),
      Wb, bb.reshape(1, -1), Wc_pad, bc_pad)


def _head_body(num_ref, den_ref, ex_ref, rw_ref, rb_ref, cw_ref, cb_ref,
               lg_ref, yh_ref, as_ref):
    den = den_ref[...][0:1, 0:1]
    hp = num_ref[...] / den
    hr = jax.nn.relu(
        jnp.dot(hp, rw_ref[...], preferred_element_type=jnp.float32)
        + rb_ref[...])
    lg = jnp.dot(hr, cw_ref[...], preferred_element_type=jnp.float32) \
        + cb_ref[...]
    lg_ref[...] = lg
    lane = lax.broadcasted_iota(jnp.int32, (1, 128), 1)
    l0 = jnp.sum(jnp.where(lane == 0, lg, 0.0))
    l1 = jnp.sum(jnp.where(lane == 1, lg, 0.0))
    yh_ref[...] = jnp.where(l1 > l0, 1, 0).astype(jnp.int32).reshape(1, 1)
    as_ref[...] = ex_ref[...] / den


def _head(num, den, exA, rW, rb, cW_pad, cb_pad):
    return pl.pallas_call(
        _head_body,
        out_shape=[
            jax.ShapeDtypeStruct((1, 128), jnp.float32),
            jax.ShapeDtypeStruct((1, 1), jnp.int32),
            jax.ShapeDtypeStruct((N, 1), jnp.float32),
        ],
    )(num, den, exA, rW, rb.reshape(1, -1), cW_pad, cb_pad)


_MESH = plsc.VectorSubcoreMesh(core_axis_name="c", subcore_axis_name="s")


def _seg_body(tabs, src_hbm, dst_hbm, outs,
              srcb0, srcb1, dstb0, dstb1, g0, g1,
              sg0, sg1, ss0, ss1, si0, si1, acc):
    c = lax.axis_index("c")
    s = lax.axis_index("s")
    srcb = [srcb0, srcb1]
    dstb = [dstb0, dstb1]
    G = [g0, g1]
    sg = [sg0, sg1]
    ss = [ss0, ss1]
    si = [si0, si1]
    src_hb = src_hbm.at[s]
    dst_hb = dst_hbm.at[s]

    def zero_acc():
        @pl.loop(0, EPB)
        def _(r):
            for j in range(128 // 16):
                g0[r, pl.ds(j * 16, 16)] = jnp.zeros((16,), jnp.float32)

        @pl.loop(0, ZR // 128)
        def _(r):
            pltpu.sync_copy(g0, acc.at[pl.ds(s * ZR + r * 128, 128)])

        pltpu.sync_copy(g0.at[pl.ds(0, ZR % 128)],
                        acc.at[pl.ds(s * ZR + (ZR // 128) * 128, ZR % 128)])

    def idx_prefetch(g, par):
        pltpu.async_copy(src_hb.at[pl.ds(g * GRP, GRP)], srcb[par], si[par])
        pltpu.async_copy(dst_hb.at[pl.ds(g * GRP, GRP)], dstb[par], si[par])

    def one_pass(k):
        tab = tabs.at[k]
        zero_acc()
        plsc.subcore_barrier()
        idx_prefetch(0, 0)

        def do_group(g, par):
            @pl.when(g + 1 < NGRP)
            def _():
                idx_prefetch(g + 1, 1 - par)

            src8, ids8 = srcb[par], dstb[par]
            pltpu.make_async_copy(src_hb.at[pl.ds(0, GRP)], src8,
                                  si[par]).wait()
            pltpu.make_async_copy(dst_hb.at[pl.ds(0, GRP)], ids8,
                                  si[par]).wait()
            pltpu.async_copy(tab.at[src8.at[0]], G[0], sg[0])
            for r in range(GRP):
                q = r % 2
                pltpu.make_async_copy(tab.at[src8.at[r]], G[q],
                                      sg[q]).wait()
                if r + 1 < GRP:
                    if r >= 1:
                        pltpu.make_async_copy(
                            G[1 - q], acc.at[ids8.at[r - 1]],
                            ss[1 - q]).wait()
                    pltpu.async_copy(tab.at[src8.at[r + 1]], G[1 - q],
                                     sg[1 - q])
                pltpu.async_copy(G[q], acc.at[ids8.at[r]], ss[q],
                                 add=True)
            pltpu.make_async_copy(G[0], acc.at[ids8.at[GRP - 2]],
                                  ss[0]).wait()
            pltpu.make_async_copy(G[1], acc.at[ids8.at[GRP - 1]],
                                  ss[1]).wait()

        @pl.loop(0, NGRP // 2)
        def _(gp):
            do_group(2 * gp, 0)
            do_group(2 * gp + 1, 1)

        plsc.subcore_barrier()
        @pl.when(s < NT - 1)
        def _():
            pltpu.sync_copy(acc.at[pl.ds(s * ZR, ZR)],
                            outs.at[k].at[pl.ds(s * ZR, ZR)])

        @pl.when(s == NT - 1)
        def _():
            pltpu.sync_copy(acc.at[pl.ds((NT - 1) * ZR, N - (NT - 1) * ZR)],
                            outs.at[k].at[pl.ds((NT - 1) * ZR,
                                                N - (NT - 1) * ZR)])

        plsc.subcore_barrier()

    @pl.when(c == 0)
    def _():
        one_pass(0)
        one_pass(1)

    @pl.when(c == 1)
    def _():
        one_pass(2)
        one_pass(3)


def _seg_sums(tabs, src_t, dst_t):
    f = pl.kernel(
        _seg_body,
        out_type=jax.ShapeDtypeStruct((4, N, 128), jnp.float32),
        mesh=_MESH,
        scratch_types=[
            pltpu.VMEM((GRP, EPB), jnp.int32),
            pltpu.VMEM((GRP, EPB), jnp.int32),
            pltpu.VMEM((GRP, EPB), jnp.int32),
            pltpu.VMEM((GRP, EPB), jnp.int32),
            pltpu.VMEM((EPB, 128), jnp.float32),
            pltpu.VMEM((EPB, 128), jnp.float32),
            pltpu.SemaphoreType.DMA,
            pltpu.SemaphoreType.DMA,
            pltpu.SemaphoreType.DMA,
            pltpu.SemaphoreType.DMA,
            pltpu.SemaphoreType.DMA,
            pltpu.SemaphoreType.DMA,
            pltpu.VMEM_SHARED((NACC, 128), jnp.float32),
        ],
    )
    return f(tabs, src_t, dst_t)


def kernel(x, params, edge_index):
    src = edge_index[0]
    dst = edge_index[1]
    pad = EPAD - E
    src_p = jnp.concatenate([src, jnp.zeros((pad,), jnp.int32)])
    dst_p = jnp.concatenate(
        [dst, N + (jnp.arange(pad, dtype=jnp.int32) % (NACC - N))])
    src_t = src_p.reshape(NT, NB, EPB)
    dst_t = dst_p.reshape(NT, NB, EPB)

    h = _fc(x, params['fc_W'], params['fc_b'])
    feats = [h]
    for li, name in enumerate(('conv0', 'conv1', 'conv2')):
        p = params[name]
        tabs = _prep(h, p['t'])
        acc = _seg_sums(tabs, src_t, dst_t)
        h = _mlp(acc[0], acc[1], acc[2], acc[3], h, p,
                 extra_norm=(li > 0))
        feats.append(h)

    x_ = jnp.concatenate(feats, axis=1)
    Wc_pad = jnp.pad(params['attn_Wc'], ((0, 0), (0, 127)))
    bc_pad = jnp.pad(params['attn_bc'], (0, 127)).reshape(1, 128)
    exA, num, den = _attn(x_, params['phi_W'], params['phi_b'],
                          params['attn_Wa'], params['attn_ba'],
                          params['attn_Wb'], params['attn_bb'],
                          Wc_pad, bc_pad)
    cW_pad = jnp.pad(params['cls_W'], ((0, 0), (0, 126)))
    cb_pad = jnp.pad(params['cls_b'], (0, 126)).reshape(1, 128)
    lg_pad, yhat, asoft_col = _head(num, den, exA, params['rho_W'],
                                    params['rho_b'], cW_pad, cb_pad)
    logits = lg_pad[:, :2]
    A_soft = asoft_col.T
    return (logits, yhat, A_soft)
